# baseline (speedup 1.0000x reference)
import functools

import jax
import jax.numpy as jnp
import numpy as np
from jax.experimental import pallas as pl
from jax.experimental.pallas import tpu as pltpu

N = 10000
E = 160000
DIN = 256
H = 8
C = 256
OUT = 11

BN = 400


def _dense1_body(x_ref, w_ref, am_ref, xp_ref, a_ref):
    xb = x_ref[...]
    xp = jnp.dot(xb, w_ref[...], preferred_element_type=jnp.float32)
    xp_ref[...] = xp
    a_ref[...] = jnp.dot(xp, am_ref[...], preferred_element_type=jnp.float32)


def _dense1(x, w_gat, att_mat):
    grid = N // BN
    return pl.pallas_call(
        _dense1_body,
        grid=(grid,),
        in_specs=[
            pl.BlockSpec((BN, DIN), lambda i: (i, 0)),
            pl.BlockSpec((DIN, H * C), lambda i: (0, 0)),
            pl.BlockSpec((H * C, 16), lambda i: (0, 0)),
        ],
        out_specs=[
            pl.BlockSpec((BN, H * C), lambda i: (i, 0)),
            pl.BlockSpec((BN, 16), lambda i: (i, 0)),
        ],
        out_shape=[
            jax.ShapeDtypeStruct((N, H * C), jnp.float32),
            jax.ShapeDtypeStruct((N, 16), jnp.float32),
        ],
    )(x, w_gat, att_mat)


def _head_body(acc_ref, bg_ref, w1_ref, b1_ref, w2_ref, b2_ref, w3_ref,
               b3_ref, w4_ref, b4_ref, w5_ref, b5_ref, out_ref):
    h = acc_ref[...] * (1.0 / H) + bg_ref[...]
    h = jnp.where(h >= 0, h, 0.01 * h)
    for w_ref, b_ref in ((w1_ref, b1_ref), (w2_ref, b2_ref),
                         (w3_ref, b3_ref), (w4_ref, b4_ref)):
        h = jnp.dot(h, w_ref[...], preferred_element_type=jnp.float32) + b_ref[...]
        h = jnp.where(h >= 0, h, 0.01 * h)
    out_ref[...] = jnp.dot(h, w5_ref[...], preferred_element_type=jnp.float32) + b5_ref[...]


def _head(acc, bg, ws, bs, w5, b5):
    grid = N // BN
    full = lambda i: (0, 0)
    in_specs = [pl.BlockSpec((BN, C), lambda i: (i, 0)),
                pl.BlockSpec((1, C), full)]
    ops = [acc, bg.reshape(1, C)]
    for w, b in zip(ws, bs):
        in_specs.append(pl.BlockSpec((C, C), full))
        in_specs.append(pl.BlockSpec((1, C), full))
        ops.append(w)
        ops.append(b.reshape(1, C))
    in_specs.append(pl.BlockSpec((C, OUT), full))
    in_specs.append(pl.BlockSpec((1, OUT), full))
    ops.append(w5)
    ops.append(b5.reshape(1, OUT))
    return pl.pallas_call(
        _head_body,
        grid=(grid,),
        in_specs=in_specs,
        out_specs=pl.BlockSpec((BN, OUT), lambda i: (i, 0)),
        out_shape=jax.ShapeDtypeStruct((N, OUT), jnp.float32),
    )(*ops)


def kernel(x, edge_index, edge_attr, params):
    p = params
    eye = jnp.eye(H, dtype=jnp.float32)
    am_src = (p['att_src'][:, None, :] * eye[:, :, None]).transpose(1, 2, 0).reshape(H * C, H)
    am_dst = (p['att_dst'][:, None, :] * eye[:, :, None]).transpose(1, 2, 0).reshape(H * C, H)
    att_mat = jnp.concatenate([am_src, am_dst], axis=1)

    xp, a = _dense1(x, p['W_gat'], att_mat)

    loops = jnp.arange(N, dtype=edge_index.dtype)
    src = jnp.concatenate([edge_index[0], loops])
    dst = jnp.concatenate([edge_index[1], loops])
    alpha = a[src, :H] + a[dst, H:]
    alpha = jnp.where(alpha >= 0, alpha, 0.2 * alpha)
    ex = jnp.exp(alpha)
    denom = jax.ops.segment_sum(ex, dst, num_segments=N)
    attn = ex / (denom[dst] + 1e-16)
    m = jnp.sum(xp[src].reshape(-1, H, C) * attn[:, :, None], axis=1)
    acc = jax.ops.segment_sum(m, dst, num_segments=N)

    ws, bs = [], []
    for i in range(1, 5):
        s = p['g%d' % i] * jax.lax.rsqrt(p['rv%d' % i] + 1e-5)
        ws.append(p['w%d' % i] * s[None, :])
        bs.append((p['b%d' % i] - p['rm%d' % i]) * s + p['be%d' % i])

    cand_ids = _head(acc, p['bias_gat'], ws, bs, p['w5'], p['b5'])

    edge_weight = edge_attr.squeeze(-1)
    cand_p4 = jnp.zeros((N, 3), jnp.float32)
    return (edge_weight, cand_ids, cand_p4)


# speedup vs baseline: 34.6581x; 34.6581x over previous
import jax
import jax.numpy as jnp
from jax import lax
from jax.experimental import pallas as pl
from jax.experimental.pallas import tpu as pltpu

N = 10000
E = 160000
DIN = 256
H = 8
C = 256
OUT = 11
HC = H * C

BN = 400
BE = 512
ETOT = E + N
EP = ((ETOT + BE - 1) // BE) * BE


def _dense1_body(x_ref, w_ref, am_ref, xp_ref, a1_ref, a2_ref):
    xb = x_ref[...]
    xp = jnp.dot(xb, w_ref[...], preferred_element_type=jnp.float32)
    xp_ref[...] = xp
    a = jnp.dot(xp, am_ref[...], preferred_element_type=jnp.float32)
    a1_ref[...] = a
    a2_ref[...] = jnp.concatenate([a[:, H:], a[:, :H]], axis=-1)


def _dense1(x, w_gat, att_mat):
    grid = N // BN
    return pl.pallas_call(
        _dense1_body,
        grid=(grid,),
        in_specs=[
            pl.BlockSpec((BN, DIN), lambda i: (i, 0)),
            pl.BlockSpec((DIN, HC), lambda i: (0, 0)),
            pl.BlockSpec((HC, 16), lambda i: (0, 0)),
        ],
        out_specs=[
            pl.BlockSpec((BN, HC), lambda i: (i, 0)),
            pl.BlockSpec((BN, 16), lambda i: (i, 0)),
            pl.BlockSpec((BN, 16), lambda i: (i, 0)),
        ],
        out_shape=[
            jax.ShapeDtypeStruct((N, HC), jnp.float32),
            jax.ShapeDtypeStruct((N, 16), jnp.float32),
            jax.ShapeDtypeStruct((N, 16), jnp.float32),
        ],
    )(x, w_gat, att_mat)


def _edge_w_body(g1_ref, g2_ref, w_ref):
    xv = g1_ref[...][:, :H] + g2_ref[...][:, :H]
    xv = jnp.where(xv >= 0.0, xv, 0.2 * xv)
    w_ref[...] = jnp.exp(xv)


def _edge_w(ga1, ga2):
    return pl.pallas_call(
        _edge_w_body,
        grid=(EP // BE,),
        in_specs=[pl.BlockSpec((BE, 16), lambda i: (i, 0)),
                  pl.BlockSpec((BE, 16), lambda i: (i, 0))],
        out_specs=pl.BlockSpec((BE, H), lambda i: (i, 0)),
        out_shape=jax.ShapeDtypeStruct((EP, H), jnp.float32),
    )(ga1, ga2)


def _edge_msg_body(w_ref, dn_ref, gxp_ref, m_ref):
    al = w_ref[...] / (dn_ref[...] + 1e-16)
    gxp = gxp_ref[...]
    m = jnp.zeros((BE, C), jnp.float32)
    for h in range(H):
        m = m + al[:, h][:, None] * gxp[:, h * C:(h + 1) * C]
    m_ref[...] = m


def _edge_msg(w, gdn, gxp):
    return pl.pallas_call(
        _edge_msg_body,
        grid=(EP // BE,),
        in_specs=[pl.BlockSpec((BE, H), lambda i: (i, 0)),
                  pl.BlockSpec((BE, H), lambda i: (i, 0)),
                  pl.BlockSpec((BE, HC), lambda i: (i, 0))],
        out_specs=pl.BlockSpec((BE, C), lambda i: (i, 0)),
        out_shape=jax.ShapeDtypeStruct((EP, C), jnp.float32),
    )(w, gdn, gxp)


def _head_body(acc_ref, bg_ref, w1_ref, b1_ref, w2_ref, b2_ref, w3_ref,
               b3_ref, w4_ref, b4_ref, w5_ref, b5_ref, out_ref):
    h = acc_ref[...] * (1.0 / H) + bg_ref[...]
    h = jnp.where(h >= 0, h, 0.01 * h)
    for w_ref, b_ref in ((w1_ref, b1_ref), (w2_ref, b2_ref),
                         (w3_ref, b3_ref), (w4_ref, b4_ref)):
        h = jnp.dot(h, w_ref[...], preferred_element_type=jnp.float32) + b_ref[...]
        h = jnp.where(h >= 0, h, 0.01 * h)
    out_ref[...] = jnp.dot(h, w5_ref[...], preferred_element_type=jnp.float32) + b5_ref[...]


def _head(acc, bg, ws, bs, w5, b5):
    grid = N // BN
    full = lambda i: (0, 0)
    in_specs = [pl.BlockSpec((BN, C), lambda i: (i, 0)),
                pl.BlockSpec((1, C), full)]
    ops = [acc, bg.reshape(1, C)]
    for w, b in zip(ws, bs):
        in_specs.append(pl.BlockSpec((C, C), full))
        in_specs.append(pl.BlockSpec((1, C), full))
        ops.append(w)
        ops.append(b.reshape(1, C))
    in_specs.append(pl.BlockSpec((C, OUT), full))
    in_specs.append(pl.BlockSpec((1, OUT), full))
    ops.append(w5)
    ops.append(b5.reshape(1, OUT))
    return pl.pallas_call(
        _head_body,
        grid=(grid,),
        in_specs=in_specs,
        out_specs=pl.BlockSpec((BN, OUT), lambda i: (i, 0)),
        out_shape=jax.ShapeDtypeStruct((N, OUT), jnp.float32),
    )(*ops)


def kernel(x, edge_index, edge_attr, params):
    p = params

    eyeH = jnp.eye(H, dtype=jnp.float32)
    m_src = (p['att_src'][:, :, None] * eyeH[:, None, :]).reshape(HC, H)
    m_dst = (p['att_dst'][:, :, None] * eyeH[:, None, :]).reshape(HC, H)
    att_mat = jnp.concatenate([m_src, m_dst], axis=1)

    xp, a1, a2 = _dense1(x, p['W_gat'], att_mat)

    loops = jnp.arange(N, dtype=edge_index.dtype)
    pads = jnp.zeros((EP - ETOT,), dtype=edge_index.dtype)
    padd = jnp.full((EP - ETOT,), N, dtype=edge_index.dtype)
    src = jnp.concatenate([edge_index[0], loops, pads])
    dst = jnp.concatenate([edge_index[1], loops, padd])

    ga1 = jnp.take(a1, src, axis=0, fill_value=0.0)
    ga2 = jnp.take(a2, dst, axis=0, fill_value=0.0)
    w = _edge_w(ga1, ga2)

    denom = jnp.zeros((N, H), jnp.float32).at[dst].add(w, mode='drop')
    gdn = jnp.take(denom, dst, axis=0, fill_value=1.0)
    gxp = jnp.take(xp, src, axis=0, fill_value=0.0)
    msgs = _edge_msg(w, gdn, gxp)

    acc = jnp.zeros((N, C), jnp.float32).at[dst].add(msgs, mode='drop')

    ws, bs = [], []
    for i in range(1, 5):
        s = p['g%d' % i] * lax.rsqrt(p['rv%d' % i] + 1e-5)
        ws.append(p['w%d' % i] * s[None, :])
        bs.append((p['b%d' % i] - p['rm%d' % i]) * s + p['be%d' % i])

    cand_ids = _head(acc, p['bias_gat'], ws, bs, p['w5'], p['b5'])

    edge_weight = edge_attr.squeeze(-1)
    cand_p4 = jnp.zeros((N, 3), jnp.float32)
    return (edge_weight, cand_ids, cand_p4)
